# baseline (speedup 1.0000x reference)
import functools

import numpy as np
import jax
import jax.numpy as jnp
from jax import lax
from jax.experimental import pallas as pl
from jax.experimental.pallas import tpu as pltpu
from jax.experimental.pallas import tpu_sc as plsc

B, S, D = 1024, 200, 64
N = B * S
NW = 32
RPT = N // NW
C = 200
NCHUNK = RPT // C
NV = D // 16
SCALE = 8.0
GSPLIT = (128, 72)


def _pos_np():
    pos = np.arange(S)[:, None].astype(np.float64)
    i = np.arange(D)[None, :].astype(np.float64)
    angle_rates = 1.0 / np.power(10000, 2 * (i // 2) / np.float32(D))
    ang = pos * angle_rates
    ang[:, 0::2] = np.sin(ang[:, 0::2])
    ang[:, 1::2] = np.cos(ang[:, 1::2])
    return ang.astype(np.float32).reshape(-1)


_POS = _pos_np()

_mesh = plsc.VectorSubcoreMesh(core_axis_name="c", subcore_axis_name="s")


@functools.partial(
    pl.kernel,
    out_type=jax.ShapeDtypeStruct((N * D,), jnp.float32),
    mesh=_mesh,
    scratch_types=[
        pltpu.VMEM((RPT,), jnp.int32),
        pltpu.VMEM((RPT + 16,), jnp.int32),
        pltpu.VMEM((S * D,), jnp.float32),
        pltpu.VMEM((C, 128), jnp.float32),
        pltpu.VMEM((C, 128), jnp.float32),
        pltpu.VMEM((C * D,), jnp.float32),
        pltpu.VMEM((C * D,), jnp.float32),
        pltpu.SemaphoreType.DMA,
        pltpu.SemaphoreType.DMA,
        pltpu.SemaphoreType.DMA,
        pltpu.SemaphoreType.DMA,
    ],
)
def _emb_kernel(pair_hbm, hoff_hbm, w_hbm, pos_hbm, out_hbm,
                pair_v, hoff_v, pos_v, pb0, pb1, ob0, ob1,
                g0, g1, w0, w1):
    pbufs = (pb0, pb1)
    obufs = (ob0, ob1)
    gsem = (g0, g1)
    wsem = (w0, w1)
    wid = lax.axis_index("s") * 2 + lax.axis_index("c")
    base = wid * RPT
    pltpu.sync_copy(pair_hbm.at[pl.ds(base, RPT)], pair_v)
    pltpu.sync_copy(hoff_hbm.at[pl.ds(base, RPT)], hoff_v.at[pl.ds(0, RPT)])
    pltpu.sync_copy(pos_hbm, pos_v)

    def fire_gathers(c):
        buf = c % 2
        copies = []
        off = 0
        for g in GSPLIT:
            copies.append(pltpu.async_copy(
                w_hbm.at[pair_v.at[pl.ds(c * C + off, g)]],
                pbufs[buf].at[pl.ds(off, g)], gsem[buf]))
            off += g
        return copies

    gathers = {0: fire_gathers(0)}
    writebacks = {}
    for c in range(NCHUNK):
        buf = c % 2
        if c + 1 < NCHUNK:
            gathers[c + 1] = fire_gathers(c + 1)
        for cp in gathers.pop(c):
            cp.wait()
        if c - 2 >= 0:
            writebacks.pop(c - 2).wait()

        pbuf = pbufs[buf]
        obuf = obufs[buf]
        cbase = c * C

        @plsc.parallel_loop(0, S, step=1, unroll=4)
        def p_body(p):
            ho = hoff_v[pl.ds(cbase + p, 16)][0]
            for d in range(NV):
                pv = pos_v[pl.ds(p * D + d * 16, 16)]
                v = pbuf[p, pl.ds(ho + d * 16, 16)]
                obuf[pl.ds(p * D + d * 16, 16)] = v * SCALE + pv

        writebacks[c] = pltpu.async_copy(
            obuf, out_hbm.at[pl.ds((base + cbase) * D, C * D)], wsem[buf])
    for c in sorted(writebacks):
        writebacks.pop(c).wait()


def kernel(x, W):
    xi = x.reshape(-1).astype(jnp.int32)
    pair = xi >> 1
    hoff = (xi & 1) * D
    pos = jnp.asarray(_POS)
    wp = W.reshape(W.shape[0] // 2, 2 * D)
    out = _emb_kernel(pair, hoff, wp, pos)
    return out.reshape(B, S, D)


# speedup vs baseline: 1.9169x; 1.9169x over previous
import functools

import numpy as np
import jax
import jax.numpy as jnp
from jax import lax
from jax.experimental import pallas as pl
from jax.experimental.pallas import tpu as pltpu
from jax.experimental.pallas import tpu_sc as plsc

B, S, D = 1024, 200, 64
N = B * S
NW = 32
RPT = N // NW
C = 200
NCHUNK = RPT // C
NV = D // 16
SCALE = 8.0
GSPLIT = (128, 72)

VOCAB = 1000000
VPAIR = VOCAB // 2
VBLK = 384
BPAIR = VBLK // 2
NFULL = VOCAB // VBLK
TAIL = VOCAB - NFULL * VBLK
IPAD = VBLK + 16


def _pos_np():
    pos = np.arange(S)[:, None].astype(np.float64)
    i = np.arange(D)[None, :].astype(np.float64)
    angle_rates = 1.0 / np.power(10000, 2 * (i // 2) / np.float32(D))
    ang = pos * angle_rates
    ang[:, 0::2] = np.sin(ang[:, 0::2])
    ang[:, 1::2] = np.cos(ang[:, 1::2])
    return ang.astype(np.float32).reshape(-1)


_POS = _pos_np()

_mesh = plsc.VectorSubcoreMesh(core_axis_name="c", subcore_axis_name="s")


@functools.partial(
    pl.kernel,
    out_type=jax.ShapeDtypeStruct((VPAIR, 128), jnp.float32),
    mesh=_mesh,
    scratch_types=[
        pltpu.VMEM((D, IPAD), jnp.float32),
        pltpu.VMEM((D, IPAD), jnp.float32),
        pltpu.VMEM((BPAIR, 128), jnp.float32),
        pltpu.VMEM((BPAIR, 128), jnp.float32),
        pltpu.SemaphoreType.DMA,
        pltpu.SemaphoreType.DMA,
        pltpu.SemaphoreType.DMA,
        pltpu.SemaphoreType.DMA,
    ],
    compiler_params=pltpu.CompilerParams(needs_layout_passes=False),
)
def _transpose_kernel(wt_hbm, tail_hbm, t8_hbm, in0, in1, ob0, ob1,
                      gi0, gi1, go0, go1):
    ins = (in0, in1)
    obs = (ob0, ob1)
    gsem = (gi0, gi1)
    wsem = (go0, go1)
    wid = lax.axis_index("s") * 2 + lax.axis_index("c")
    iota = lax.broadcasted_iota(jnp.int32, (16,), 0)

    def fire_in(bi, j):
        return pltpu.async_copy(
            wt_hbm.at[:, pl.ds(bi * VBLK, VBLK)],
            ins[j].at[:, pl.ds(0, VBLK)], gsem[j])

    def drain_in(j):
        pltpu.make_async_copy(
            wt_hbm.at[:, pl.ds(0, VBLK)],
            ins[j].at[:, pl.ds(0, VBLK)], gsem[j]).wait()

    pj_pat = (iota >> 1, (iota + 1) >> 1)
    lane_pat = ((iota & 1) * D + iota, ((iota + 1) & 1) * D + iota)

    def do_block(j):
        ib = ins[j]
        ob = obs[j]

        def diag(u, par, mask):
            rl = u + iota
            idx_p = pj_pat[par] + ((u - par) >> 1)
            for l in range(NV):
                cvec = iota + l * 16
                col = plsc.load_gather(ib, [cvec, rl], mask=mask)
                plsc.store_scatter(ob, [idx_p, lane_pat[par] + l * 16],
                                   col, mask=mask)

        @plsc.parallel_loop(0, 184, step=1, unroll=1)
        def mid(t):
            diag(2 * t, 0, None)
            diag(2 * t + 1, 1, None)

        diag(368, 0, None)

        @plsc.parallel_loop(0, 15, step=1, unroll=2)
        def edges(e):
            for base in (e - 15, e + 369):
                rl = base + iota
                mask = (rl >= 0) & (rl < VBLK)
                rlc = jnp.clip(rl, 0, VBLK - 1)
                idx_p = rlc >> 1
                lane_half = (rlc & 1) * D
                for l in range(NV):
                    cvec = iota + l * 16
                    col = plsc.load_gather(ib, [cvec, rlc], mask=mask)
                    plsc.store_scatter(ob, [idx_p, lane_half + cvec],
                                       col, mask=mask)

    def wait_out(j):
        pltpu.make_async_copy(
            obs[j], t8_hbm.at[pl.ds(0, BPAIR)], wsem[j]).wait()

    def blk_body(k2, carry):
        for j in range(2):
            bi = wid + 32 * (2 * k2 + j)

            @pl.when(bi < NFULL)
            def _fire():
                fire_in(bi, j)
        for j in range(2):
            bi = wid + 32 * (2 * k2 + j)

            @pl.when(bi < NFULL)
            def _do():
                drain_in(j)

                @pl.when(k2 > 0)
                def _wo():
                    wait_out(j)

                do_block(j)
                pltpu.async_copy(obs[j], t8_hbm.at[pl.ds(bi * BPAIR, BPAIR)],
                                 wsem[j])
        return carry

    lax.fori_loop(0, 41, blk_body, 0)
    wait_out(0)

    @pl.when(wid < 12)
    def _wo1():
        wait_out(1)

    @pl.when(wid == 0)
    def _tail():
        pltpu.sync_copy(tail_hbm, ob0.at[pl.ds(0, TAIL // 2)])
        pltpu.sync_copy(ob0.at[pl.ds(0, TAIL // 2)],
                        t8_hbm.at[pl.ds(NFULL * VBLK // 2, TAIL // 2)])


@functools.partial(
    pl.kernel,
    out_type=jax.ShapeDtypeStruct((B, S, D), jnp.float32),
    mesh=_mesh,
    scratch_types=[
        pltpu.VMEM((RPT,), jnp.int32),
        pltpu.VMEM((RPT + 16,), jnp.int32),
        pltpu.VMEM((S * D,), jnp.float32),
        pltpu.VMEM((C, 128), jnp.float32),
        pltpu.VMEM((C, 128), jnp.float32),
        pltpu.VMEM((C, D), jnp.float32),
        pltpu.VMEM((C, D), jnp.float32),
        pltpu.SemaphoreType.DMA,
        pltpu.SemaphoreType.DMA,
        pltpu.SemaphoreType.DMA,
        pltpu.SemaphoreType.DMA,
    ],
)
def _emb_kernel(pair_hbm, hoff_hbm, t8_hbm, pos_hbm, out_hbm,
                pair_v, hoff_v, pos_v, pb0, pb1, ob0, ob1,
                g0, g1, w0, w1):
    pbufs = (pb0, pb1)
    obufs = (ob0, ob1)
    gsem = (g0, g1)
    wsem = (w0, w1)
    wid = lax.axis_index("s") * 2 + lax.axis_index("c")
    base = wid * RPT
    b0 = wid * (B // NW)
    pltpu.sync_copy(pair_hbm.at[pl.ds(base, RPT)], pair_v)
    pltpu.sync_copy(hoff_hbm.at[pl.ds(base, RPT)], hoff_v.at[pl.ds(0, RPT)])
    pltpu.sync_copy(pos_hbm, pos_v)

    def fire_gathers(c):
        buf = c % 2
        copies = []
        off = 0
        for g in GSPLIT:
            copies.append(pltpu.async_copy(
                t8_hbm.at[pair_v.at[pl.ds(c * C + off, g)]],
                pbufs[buf].at[pl.ds(off, g)], gsem[buf]))
            off += g
        return copies

    gathers = {0: fire_gathers(0)}
    writebacks = {}
    for c in range(NCHUNK):
        buf = c % 2
        if c + 1 < NCHUNK:
            gathers[c + 1] = fire_gathers(c + 1)
        for cp in gathers.pop(c):
            cp.wait()
        if c - 2 >= 0:
            writebacks.pop(c - 2).wait()

        pbuf = pbufs[buf]
        obuf = obufs[buf]
        cbase = c * C

        @plsc.parallel_loop(0, S, step=1, unroll=4)
        def p_body(p):
            ho = hoff_v[pl.ds(cbase + p, 16)][0]
            for d in range(NV):
                pv = pos_v[pl.ds(p * D + d * 16, 16)]
                v = pbuf[p, pl.ds(ho + d * 16, 16)]
                obuf[p, pl.ds(d * 16, 16)] = v * SCALE + pv

        writebacks[c] = pltpu.async_copy(
            obuf, out_hbm.at[b0 + c], wsem[buf])
    for c in sorted(writebacks):
        writebacks.pop(c).wait()


def kernel(x, W):
    xi = x.reshape(-1).astype(jnp.int32)
    pair = xi >> 1
    hoff = (xi & 1) * D
    pos = jnp.asarray(_POS)
    tail8 = W[NFULL * VBLK:, :].reshape(TAIL // 2, 2 * D)
    t8 = _transpose_kernel(W.T, tail8)
    return _emb_kernel(pair, hoff, t8, pos)
